# baseline (speedup 1.0000x reference)
import dataclasses
import functools

import jax
import jax.numpy as jnp
from jax import lax
from jax.experimental import pallas as pl
from jax.experimental.pallas import tpu as pltpu
from jax.experimental.pallas import tpu_sc as plsc

R, C = 128, 32768
K = 64
L = 16
NWORKERS = 32
ROWS_PER = R // NWORKERS

H1_BITS, H2_BITS, H3_BITS = 12, 12, 7
H1, H2, H3 = 1 << H1_BITS, 1 << H2_BITS, 1 << H3_BITS


def _zero_hist(h, nbins):
    zeros = jnp.zeros((L,), jnp.int32)

    @plsc.parallel_loop(0, nbins, L, unroll=8)
    def _(i):
        h[pl.ds(i, L)] = zeros


def _scan_hist(h, nbins, rank):
    nv = nbins // L

    def cond(st):
        _, cum = st
        return cum < rank

    def body(st):
        v, cum = st
        s = jnp.sum(h[pl.ds(v * L, L)])
        return (v - 1, cum + s)

    v_end, cum = lax.while_loop(cond, body,
                                (jnp.int32(nv - 1), jnp.int32(0)))
    vstar = v_end + 1
    hv = h[pl.ds(vstar * L, L)]
    s = jnp.sum(hv)
    prev = cum - s
    rev = lax.rev(hv, (0,))
    csum = plsc.cumsum(rev)
    need = rank - prev
    lane_v = plsc.all_reduce_ffs(csum >= need)
    li = lax.iota(jnp.int32, L)
    lane = jnp.sum(jnp.where(li == lane_v, li, 0))
    c_at = jnp.sum(jnp.where(li == lane, csum, 0))
    h_at = jnp.sum(jnp.where(li == lane, rev, 0))
    bin_index = vstar * L + (L - 1 - lane)
    above = prev + c_at - h_at
    return bin_index, rank - above


def kernel(x):
    mesh = plsc.VectorSubcoreMesh(core_axis_name="c", subcore_axis_name="s")
    cp = pltpu.CompilerParams()
    if "needs_layout_passes" in pltpu.CompilerParams.__dataclass_fields__:
        cp = dataclasses.replace(cp, needs_layout_passes=False)

    @functools.partial(
        pl.kernel,
        out_type=jax.ShapeDtypeStruct((R, C), jnp.float32),
        mesh=mesh,
        compiler_params=cp,
        scratch_types=[
            pltpu.VMEM((C,), jnp.float32),
            pltpu.VMEM((C,), jnp.float32),
            pltpu.VMEM((C,), jnp.int32),
            pltpu.VMEM((H1,), jnp.int32),
            pltpu.VMEM((H2,), jnp.int32),
            pltpu.VMEM((H3,), jnp.int32),
        ],
    )
    def k(x_hbm, o_hbm, row_v, zbuf, cand, h1, h2, h3):
        wid = lax.axis_index("s") * 2 + lax.axis_index("c")

        ones = jnp.ones((L,), jnp.int32)
        zeros_i = jnp.zeros((L,), jnp.int32)
        zeros_f = jnp.zeros((L,), jnp.float32)
        li = lax.iota(jnp.int32, L)

        @plsc.parallel_loop(0, C, L, unroll=8)
        def _(i):
            zbuf[pl.ds(i, L)] = zeros_f

        _zero_hist(h1, H1)
        _zero_hist(h2, H2)
        _zero_hist(h3, H3)

        def row_body(j, b_spec):
            row = wid * ROWS_PER + j
            pltpu.sync_copy(x_hbm.at[row], row_v)

            def body_a(v, base):
                i = v * L
                r = jnp.maximum(row_v[pl.ds(i, L)], 0.0)
                kk = plsc.bitcast(r, jnp.int32)
                hi = jnp.right_shift(kk, H2_BITS + H3_BITS)
                plsc.addupdate_scatter(h1, [hi], ones)
                m = hi >= b_spec
                mi = m.astype(jnp.int32)
                off = plsc.cumsum(mi) - mi
                plsc.store_scatter(cand, [base + off], i + li, mask=m)
                return base + jnp.sum(mi)

            n_spec = lax.fori_loop(0, C // L, body_a, jnp.int32(0), unroll=8)

            b1, rank2 = _scan_hist(h1, H1, jnp.int32(K))

            _zero_hist(h1, H1)

            def body_fb(v, base):
                i = v * L
                r = jnp.maximum(row_v[pl.ds(i, L)], 0.0)
                kk = plsc.bitcast(r, jnp.int32)
                m = jnp.right_shift(kk, H2_BITS + H3_BITS) >= b1
                mi = m.astype(jnp.int32)
                off = plsc.cumsum(mi) - mi
                plsc.store_scatter(cand, [base + off], i + li, mask=m)
                return base + jnp.sum(mi)

            ok = b_spec <= b1
            nfb = jnp.where(ok, jnp.int32(0), jnp.int32(C // L))
            n_fb = lax.fori_loop(0, nfb, body_fb, jnp.int32(0))
            n_cand = jnp.where(ok, n_spec, n_fb)

            ntrips = lax.shift_right_logical(n_cand + (L - 1), 4)

            def body_h2(t, _):
                i = t * L
                valid = (i + li) < n_cand
                idx = cand[pl.ds(i, L)]
                r = plsc.load_gather(row_v, [idx], mask=valid)
                kk = plsc.bitcast(jnp.maximum(r, 0.0), jnp.int32)
                m = jnp.logical_and(
                    valid, jnp.right_shift(kk, H2_BITS + H3_BITS) == b1)
                plsc.addupdate_scatter(
                    h2,
                    [jnp.bitwise_and(jnp.right_shift(kk, H3_BITS), H2 - 1)],
                    ones, mask=m)
                return _

            lax.fori_loop(0, ntrips, body_h2, jnp.int32(0))

            b2, rank3 = _scan_hist(h2, H2, rank2)
            prefix = jnp.bitwise_or(lax.shift_left(b1, H2_BITS), b2)

            def body_h3(t, _):
                i = t * L
                valid = (i + li) < n_cand
                idx = cand[pl.ds(i, L)]
                r = plsc.load_gather(row_v, [idx], mask=valid)
                kk = plsc.bitcast(jnp.maximum(r, 0.0), jnp.int32)
                m = jnp.logical_and(
                    valid, jnp.right_shift(kk, H3_BITS) == prefix)
                plsc.addupdate_scatter(
                    h3, [jnp.bitwise_and(kk, H3 - 1)], ones, mask=m)
                return _

            lax.fori_loop(0, ntrips, body_h3, jnp.int32(0))

            b3, rank_eq = _scan_hist(h3, H3, rank3)
            tkey = jnp.bitwise_or(lax.shift_left(prefix, H3_BITS), b3)

            def body_o(t, eqcount):
                i = t * L
                valid = (i + li) < n_cand
                idx = cand[pl.ds(i, L)]
                r = plsc.load_gather(row_v, [idx], mask=valid)
                r = jnp.maximum(r, 0.0)
                kk = plsc.bitcast(r, jnp.int32)
                meq = jnp.logical_and(valid, kk == tkey)
                meqi = meq.astype(jnp.int32)
                pc = plsc.cumsum(meqi)
                occ = eqcount + pc - meqi
                keep = jnp.logical_and(
                    valid,
                    jnp.logical_or(kk > tkey,
                                   jnp.logical_and(meq, occ < rank_eq)))
                plsc.store_scatter(zbuf, [idx], r, mask=keep)
                return eqcount + jnp.sum(meqi)

            lax.fori_loop(0, ntrips, body_o, jnp.int32(0))

            pltpu.sync_copy(zbuf, o_hbm.at[row])

            def body_z(t, _):
                i = t * L
                valid = (i + li) < n_cand
                idx = cand[pl.ds(i, L)]
                r = plsc.load_gather(row_v, [idx], mask=valid)
                kk = plsc.bitcast(jnp.maximum(r, 0.0), jnp.int32)
                plsc.store_scatter(zbuf, [idx], zeros_f, mask=valid)
                m2 = jnp.logical_and(
                    valid, jnp.right_shift(kk, H2_BITS + H3_BITS) == b1)
                plsc.store_scatter(
                    h2,
                    [jnp.bitwise_and(jnp.right_shift(kk, H3_BITS), H2 - 1)],
                    zeros_i, mask=m2)
                m3 = jnp.logical_and(
                    valid, jnp.right_shift(kk, H3_BITS) == prefix)
                plsc.store_scatter(
                    h3, [jnp.bitwise_and(kk, H3 - 1)], zeros_i, mask=m3)
                return _

            lax.fori_loop(0, ntrips, body_z, jnp.int32(0))

            return jnp.maximum(b1 - 1, 0)

        lax.fori_loop(0, ROWS_PER, row_body, jnp.int32(H1))

    return k(x)
